# baseline (speedup 1.0000x reference)
import jax
import jax.numpy as jnp
from jax.experimental import pallas as pl
from jax.experimental.pallas import tpu as pltpu


_CHUNK_ROWS = (256, 2048, 5888, 5888, 2048, 256)
_N_CHUNKS = len(_CHUNK_ROWS)


def _copy_body(x_hbm, o_hbm, vmem, in_sems, out_sems):
    offs = [sum(_CHUNK_ROWS[:i]) for i in range(_N_CHUNKS)]
    ins, outs = [], []
    for i in range(_N_CHUNKS):
        c = pltpu.make_async_copy(
            x_hbm.at[pl.ds(offs[i], _CHUNK_ROWS[i]), :],
            vmem.at[pl.ds(offs[i], _CHUNK_ROWS[i]), :],
            in_sems.at[i],
        )
        c.start()
        ins.append(c)
    for i in range(_N_CHUNKS):
        ins[i].wait()
        c = pltpu.make_async_copy(
            vmem.at[pl.ds(offs[i], _CHUNK_ROWS[i]), :],
            o_hbm.at[pl.ds(offs[i], _CHUNK_ROWS[i]), :],
            out_sems.at[i],
        )
        c.start()
        outs.append(c)
    for c in outs:
        c.wait()


def kernel(x):
    return pl.pallas_call(
        _copy_body,
        out_shape=jax.ShapeDtypeStruct(x.shape, x.dtype),
        in_specs=[pl.BlockSpec(memory_space=pl.MemorySpace.ANY)],
        out_specs=pl.BlockSpec(memory_space=pl.MemorySpace.ANY),
        scratch_shapes=[
            pltpu.VMEM(x.shape, x.dtype),
            pltpu.SemaphoreType.DMA((_N_CHUNKS,)),
            pltpu.SemaphoreType.DMA((_N_CHUNKS,)),
        ],
    )(x)


# speedup vs baseline: 1.0102x; 1.0102x over previous
import jax
import jax.numpy as jnp
from jax.experimental import pallas as pl
from jax.experimental.pallas import tpu as pltpu


_CHUNK_ROWS = (512, 1536, 4096, 4096, 4096, 1536, 512)
_N_CHUNKS = len(_CHUNK_ROWS)


def _copy_body(x_hbm, o_hbm, vmem, in_sems, out_sems):
    offs = [sum(_CHUNK_ROWS[:i]) for i in range(_N_CHUNKS)]
    ins, outs = [], []
    for i in range(_N_CHUNKS):
        c = pltpu.make_async_copy(
            x_hbm.at[pl.ds(offs[i], _CHUNK_ROWS[i]), :],
            vmem.at[pl.ds(offs[i], _CHUNK_ROWS[i]), :],
            in_sems.at[i],
        )
        c.start()
        ins.append(c)
    for i in range(_N_CHUNKS):
        ins[i].wait()
        c = pltpu.make_async_copy(
            vmem.at[pl.ds(offs[i], _CHUNK_ROWS[i]), :],
            o_hbm.at[pl.ds(offs[i], _CHUNK_ROWS[i]), :],
            out_sems.at[i],
        )
        c.start()
        outs.append(c)
    for c in outs:
        c.wait()


def kernel(x):
    return pl.pallas_call(
        _copy_body,
        out_shape=jax.ShapeDtypeStruct(x.shape, x.dtype),
        in_specs=[pl.BlockSpec(memory_space=pl.MemorySpace.ANY)],
        out_specs=pl.BlockSpec(memory_space=pl.MemorySpace.ANY),
        scratch_shapes=[
            pltpu.VMEM(x.shape, x.dtype),
            pltpu.SemaphoreType.DMA((_N_CHUNKS,)),
            pltpu.SemaphoreType.DMA((_N_CHUNKS,)),
        ],
    )(x)
